# baseline (speedup 1.0000x reference)
import functools

import jax
import jax.numpy as jnp
from jax import lax
from jax.experimental import pallas as pl
from jax.experimental.pallas import tpu as pltpu
from jax.experimental.pallas import tpu_sc as plsc

B, S, H = 1, 2048, 1024
E = 8
I = 512
ISH = 1024
T = B * S
K = 2

BTG = 256
NTILES = (2 * T) // BTG + (E - 1)
NBUF = NTILES * BTG

NW = 32
PPW = 2 * T // NW
RPW = 4
PPR = PPW // RPW
TPR = PPR // 2

_BS = pl.BlockSpec


def _dot_nt(a, b):
    return lax.dot_general(a, b, (((1,), (1,)), ((), ())),
                           preferred_element_type=jnp.float32)


def _route_body(x_ref, gw_ref, dest_ref, pw_ref, te_ref, valid_ref, xb_ref):
    x = x_ref[...]
    xb_ref[...] = x.astype(jnp.bfloat16)
    logits = _dot_nt(x, gw_ref[...])
    m = jnp.max(logits, axis=-1, keepdims=True)
    ex = jnp.exp(logits - m)
    scores = ex / jnp.sum(ex, axis=-1, keepdims=True)
    eidx = lax.broadcasted_iota(jnp.int32, (T, E), 1)
    i1 = jnp.argmax(scores, axis=-1)
    m1 = jnp.max(scores, axis=-1)
    oh1 = (eidx == i1[:, None]).astype(jnp.float32)
    masked = jnp.where(oh1 > 0, -jnp.inf, scores)
    i2 = jnp.argmax(masked, axis=-1)
    m2 = jnp.max(masked, axis=-1)
    oh2 = (eidx == i2[:, None]).astype(jnp.float32)
    denom = m1 + m2 + 1e-20
    pw_ref[...] = jnp.concatenate(
        [(m1 / denom)[:, None], (m2 / denom)[:, None]], axis=1)

    sel = oh1 + oh2
    r_i = lax.broadcasted_iota(jnp.int32, (BTG, BTG), 0)
    c_i = lax.broadcasted_iota(jnp.int32, (BTG, BTG), 1)
    tril = (r_i > c_i).astype(jnp.float32)
    base = jnp.zeros((1, E), jnp.float32)
    chunks = []
    for ch in range(T // BTG):
        blk = sel[ch * BTG:(ch + 1) * BTG]
        chunks.append(jnp.dot(tril, blk,
                              preferred_element_type=jnp.float32) + base)
        base = base + jnp.sum(blk, axis=0, keepdims=True)
    rank = jnp.concatenate(chunks, axis=0)
    counts = base
    pc = jnp.floor((counts + (BTG - 1)) / BTG) * BTG
    upper = (lax.broadcasted_iota(jnp.int32, (E, E), 0)
             < lax.broadcasted_iota(jnp.int32, (E, E), 1)).astype(jnp.float32)
    off = jnp.dot(pc, upper, preferred_element_type=jnp.float32)
    slot = off + rank
    d0 = jnp.sum(oh1 * slot, axis=1, keepdims=True)
    d1 = jnp.sum(oh2 * slot, axis=1, keepdims=True)
    dest_ref[...] = jnp.concatenate([d0, d1], axis=1).astype(jnp.int32)

    jv = (lax.broadcasted_iota(jnp.int32, (NTILES, 1), 0)
          * BTG).astype(jnp.float32)
    owner = jnp.sum((off <= jv).astype(jnp.float32), axis=1,
                    keepdims=True) - 1.0
    te_ref[...] = jnp.clip(owner, 0, E - 1).astype(jnp.int32)
    total = jnp.sum(pc)
    valid_ref[...] = (jv < total).astype(jnp.int32)


def _route(x, gate_w):
    return pl.pallas_call(
        _route_body,
        grid=(1,),
        in_specs=[_BS((T, H), lambda i: (0, 0)),
                  _BS((E, H), lambda i: (0, 0))],
        out_specs=[_BS((T, K), lambda i: (0, 0)),
                   _BS((T, K), lambda i: (0, 0)),
                   _BS((NTILES, 1), lambda i: (0, 0)),
                   _BS((NTILES, 1), lambda i: (0, 0)),
                   _BS((T, H), lambda i: (0, 0))],
        out_shape=[jax.ShapeDtypeStruct((T, K), jnp.int32),
                   jax.ShapeDtypeStruct((T, K), jnp.float32),
                   jax.ShapeDtypeStruct((NTILES, 1), jnp.int32),
                   jax.ShapeDtypeStruct((NTILES, 1), jnp.int32),
                   jax.ShapeDtypeStruct((T, H), jnp.bfloat16)],
    )(x, gate_w)


def _shared_body(x_ref, shg_ref, shu_ref, shd_ref, ys_ref):
    xb = x_ref[...].astype(jnp.bfloat16)
    sg = _dot_nt(xb, shg_ref[...].astype(jnp.bfloat16))
    su = _dot_nt(xb, shu_ref[...].astype(jnp.bfloat16))
    act = (sg * lax.logistic(sg)) * su
    ys_ref[...] = _dot_nt(act.astype(jnp.bfloat16),
                          shd_ref[...].astype(jnp.bfloat16))


def _shared(x, sh_gate, sh_up, sh_down):
    bt = 512
    return pl.pallas_call(
        _shared_body,
        grid=(T // bt,),
        in_specs=[_BS((bt, H), lambda i: (i, 0)),
                  _BS((ISH, H), lambda i: (0, 0)),
                  _BS((ISH, H), lambda i: (0, 0)),
                  _BS((H, ISH), lambda i: (0, 0))],
        out_specs=_BS((bt, H), lambda i: (i, 0)),
        out_shape=jax.ShapeDtypeStruct((T, H), jnp.float32),
    )(x, sh_gate, sh_up, sh_down)


def _sc_scatter_body(x_hbm, dest_hbm, xs_hbm, idx_v, dst_v, gb0, gb1, sg, ss):
    w = lax.axis_index("s") * 2 + lax.axis_index("c")
    pltpu.sync_copy(dest_hbm.at[w], dst_v)
    iota = lax.iota(jnp.int32, 16)
    gbufs = [gb0, gb1]
    for r in range(RPW):
        p0 = w * PPW + r * PPR
        idx_v[r, pl.ds(0, 16)] = (p0 + iota) >> 1
        idx_v[r, pl.ds(16, 16)] = (p0 + 16 + iota) >> 1
    gathers = [None] * RPW
    scatters = [None] * RPW
    gathers[0] = pltpu.async_copy(x_hbm.at[idx_v.at[0]], gbufs[0], sg)
    for r in range(RPW):
        gathers[r].wait()
        if r >= 1:
            scatters[r - 1].wait()
        if r + 1 < RPW:
            gathers[r + 1] = pltpu.async_copy(
                x_hbm.at[idx_v.at[r + 1]], gbufs[(r + 1) % 2], sg)
        scatters[r] = pltpu.async_copy(gbufs[r % 2],
                                       xs_hbm.at[dst_v.at[r]], ss)
    scatters[RPW - 1].wait()


def _sc_scatter(xb3, dest3):
    mesh = plsc.VectorSubcoreMesh(core_axis_name="c", subcore_axis_name="s")
    return pl.kernel(
        _sc_scatter_body,
        out_type=jax.ShapeDtypeStruct((NBUF, H // 2), jnp.int32),
        mesh=mesh,
        scratch_types=[pltpu.VMEM((RPW, PPR), jnp.int32),
                       pltpu.VMEM((RPW, PPR), jnp.int32),
                       pltpu.VMEM((PPR, H // 2), jnp.int32),
                       pltpu.VMEM((PPR, H // 2), jnp.int32),
                       pltpu.SemaphoreType.DMA,
                       pltpu.SemaphoreType.DMA],
    )(xb3, dest3)


def _gmm_body(te_ref, valid_ref, xs_ref, wg_ref, wu_ref, wd_ref, ob_ref):
    j = pl.program_id(0)

    @pl.when(valid_ref[j] > 0)
    def _():
        xb = xs_ref[...].astype(jnp.bfloat16)
        wg = wg_ref[0].astype(jnp.bfloat16)
        wu = wu_ref[0].astype(jnp.bfloat16)
        wd = wd_ref[0].astype(jnp.bfloat16)
        g = _dot_nt(xb, wg)
        u = _dot_nt(xb, wu)
        a = (g * lax.logistic(g)) * u
        ob_ref[...] = _dot_nt(a.astype(jnp.bfloat16), wd)


def _gmm(te, valid, xs, w_gate, w_up, w_down):
    grid_spec = pltpu.PrefetchScalarGridSpec(
        num_scalar_prefetch=2,
        grid=(NTILES,),
        in_specs=[_BS((BTG, H), lambda j, te, va: (j, 0)),
                  _BS((1, I, H), lambda j, te, va: (te[j], 0, 0)),
                  _BS((1, I, H), lambda j, te, va: (te[j], 0, 0)),
                  _BS((1, H, I), lambda j, te, va: (te[j], 0, 0))],
        out_specs=_BS((BTG, H), lambda j, te, va: (j, 0)),
    )
    return pl.pallas_call(
        _gmm_body,
        grid_spec=grid_spec,
        out_shape=jax.ShapeDtypeStruct((NBUF, H), jnp.float32),
    )(te, valid, xs, w_gate, w_up, w_down)


def _sc_combine_body(ob_hbm, ys_hbm, dest_hbm, pw_hbm, y_hbm,
                     dst_v, pw_v, rb0, rb1, ys_v, out_v, sg, so):
    w = lax.axis_index("s") * 2 + lax.axis_index("c")
    pltpu.sync_copy(dest_hbm.at[w], dst_v)
    pltpu.sync_copy(pw_hbm.at[w], pw_v)
    rbufs = [rb0, rb1]
    gathers = [None] * RPW
    tb = [w * (T // NW) + r * TPR for r in range(RPW)]
    gathers[0] = pltpu.async_copy(ob_hbm.at[dst_v.at[0]], rbufs[0], sg)
    stores = [None] * RPW
    for r in range(RPW):
        if r + 1 < RPW:
            gathers[r + 1] = pltpu.async_copy(
                ob_hbm.at[dst_v.at[r + 1]], rbufs[(r + 1) % 2], sg)
        pltpu.sync_copy(ys_hbm.at[pl.ds(tb[r], TPR)], ys_v)
        gathers[r].wait()
        rows_v = rbufs[r % 2]
        if r >= 2:
            stores[r - 2].wait()

        def tok_body(i, _):
            grp = (i // 8) * 16
            pv = pw_v[pl.ds(r * PPR + grp, 16)]
            lane = jnp.zeros((16,), jnp.int32) + ((2 * i) % 16)
            w0 = pv.at[lane].get(mode="promise_in_bounds")
            w1 = pv.at[lane + 1].get(mode="promise_in_bounds")
            for cc in range(H // 16):
                slc = pl.ds(cc * 16, 16)
                out_v[r % 2, i, slc] = (ys_v[i, slc]
                                        + w0 * rows_v[2 * i, slc]
                                        + w1 * rows_v[2 * i + 1, slc])
            return 0

        lax.fori_loop(0, TPR, tok_body, 0)
        stores[r] = pltpu.async_copy(out_v.at[r % 2],
                                     y_hbm.at[pl.ds(tb[r], TPR)], so)
    stores[RPW - 2].wait()
    stores[RPW - 1].wait()


def _sc_combine(ob, ys, dest3, pwf):
    mesh = plsc.VectorSubcoreMesh(core_axis_name="c", subcore_axis_name="s")
    return pl.kernel(
        _sc_combine_body,
        out_type=jax.ShapeDtypeStruct((T, H), jnp.float32),
        mesh=mesh,
        scratch_types=[pltpu.VMEM((RPW, PPR), jnp.int32),
                       pltpu.VMEM((PPW,), jnp.float32),
                       pltpu.VMEM((PPR, H), jnp.float32),
                       pltpu.VMEM((PPR, H), jnp.float32),
                       pltpu.VMEM((TPR, H), jnp.float32),
                       pltpu.VMEM((2, TPR, H), jnp.float32),
                       pltpu.SemaphoreType.DMA,
                       pltpu.SemaphoreType.DMA],
    )(ob, ys, dest3, pwf)


def kernel(hidden_states, gate_w, w_gate, w_up, w_down, sh_gate, sh_up, sh_down):
    x = hidden_states.reshape(T, H)
    dest, pw, te, valid, xb = _route(x, gate_w)
    ys = _shared(x, sh_gate, sh_up, sh_down)
    dest3 = dest.reshape(NW, RPW, PPR)
    pwf = pw.reshape(NW, PPW)
    xb_i32 = lax.bitcast_convert_type(xb.reshape(T, H // 2, 2), jnp.int32)
    xs_i32 = _sc_scatter(xb_i32, dest3)
    xs = lax.bitcast_convert_type(xs_i32, jnp.bfloat16).reshape(NBUF, H)
    ob = _gmm(te.reshape(NTILES), valid.reshape(NTILES), xs,
              w_gate, w_up, w_down)
    y = _sc_combine(ob, ys, dest3, pwf)
    return y.reshape(hidden_states.shape)


# speedup vs baseline: 3.3418x; 3.3418x over previous
import jax
import jax.numpy as jnp
from jax.experimental import pallas as pl
from jax.experimental.pallas import tpu as pltpu

B, S, H = 1, 2048, 1024
E = 8
I = 512
ISH = 1024
T = B * S
BT = 256
NTB = T // BT


def _dot_nt(a, b):
    return jax.lax.dot_general(a, b, (((1,), (1,)), ((), ())),
                               preferred_element_type=jnp.float32)


def _moe_kernel(x_ref, gatew_ref, wg_ref, wu_ref, wd_ref,
                shg_ref, shu_ref, shd_ref, out_ref, comb_ref):
    e = pl.program_id(0)

    @pl.when(e == 0)
    def _gate_and_shared():
        for tb in range(NTB):
            sl = pl.ds(tb * BT, BT)
            xb = x_ref[sl, :]
            logits = _dot_nt(xb, gatew_ref[...])
            m = jnp.max(logits, axis=-1, keepdims=True)
            ex = jnp.exp(logits - m)
            scores = ex / jnp.sum(ex, axis=-1, keepdims=True)
            eidx = jax.lax.broadcasted_iota(jnp.int32, (BT, E), 1)
            i1 = jnp.argmax(scores, axis=-1)
            m1 = jnp.max(scores, axis=-1)
            masked = jnp.where(eidx == i1[:, None], -jnp.inf, scores)
            i2 = jnp.argmax(masked, axis=-1)
            m2 = jnp.max(masked, axis=-1)
            denom = m1 + m2 + 1e-20
            w1 = (m1 / denom)[:, None]
            w2 = (m2 / denom)[:, None]
            comb = (jnp.where(eidx == i1[:, None], w1, 0.0)
                    + jnp.where(eidx == i2[:, None], w2, 0.0))
            comb_ref[sl, :] = comb
            xb16 = xb.astype(jnp.bfloat16)
            sg = _dot_nt(xb16, shg_ref[...].astype(jnp.bfloat16))
            su = _dot_nt(xb16, shu_ref[...].astype(jnp.bfloat16))
            act = (sg * jax.lax.logistic(sg)) * su
            ys = _dot_nt(act.astype(jnp.bfloat16),
                         shd_ref[...].astype(jnp.bfloat16))
            out_ref[sl, :] = ys

    @pl.when(e > 0)
    def _routed():
        wg = wg_ref[0].astype(jnp.bfloat16)
        wu = wu_ref[0].astype(jnp.bfloat16)
        wd = wd_ref[0].astype(jnp.bfloat16)
        onehot = (jax.lax.broadcasted_iota(jnp.int32, (E, 1), 0)
                  == e - 1).astype(jnp.float32)
        for tb in range(NTB):
            sl = pl.ds(tb * BT, BT)
            xb16 = x_ref[sl, :].astype(jnp.bfloat16)
            g = _dot_nt(xb16, wg)
            u = _dot_nt(xb16, wu)
            a = (g * jax.lax.logistic(g)) * u
            eo = _dot_nt(a.astype(jnp.bfloat16), wd)
            cw = jnp.dot(comb_ref[sl, :], onehot,
                         preferred_element_type=jnp.float32)
            out_ref[sl, :] += cw * eo


def kernel(hidden_states, gate_w, w_gate, w_up, w_down, sh_gate, sh_up, sh_down):
    x = hidden_states.reshape(T, H)
    out = pl.pallas_call(
        _moe_kernel,
        grid=(E + 1,),
        in_specs=[
            pl.BlockSpec((T, H), lambda e: (0, 0)),
            pl.BlockSpec((E, H), lambda e: (0, 0)),
            pl.BlockSpec((1, I, H), lambda e: (jnp.maximum(e - 1, 0), 0, 0)),
            pl.BlockSpec((1, I, H), lambda e: (jnp.maximum(e - 1, 0), 0, 0)),
            pl.BlockSpec((1, H, I), lambda e: (jnp.maximum(e - 1, 0), 0, 0)),
            pl.BlockSpec((ISH, H), lambda e: (0, 0)),
            pl.BlockSpec((ISH, H), lambda e: (0, 0)),
            pl.BlockSpec((H, ISH), lambda e: (0, 0)),
        ],
        out_specs=pl.BlockSpec((T, H), lambda e: (0, 0)),
        out_shape=jax.ShapeDtypeStruct((T, H), jnp.float32),
        scratch_shapes=[pltpu.VMEM((T, E), jnp.float32)],
    )(x, gate_w, w_gate, w_up, w_down, sh_gate, sh_up, sh_down)
    return out.reshape(hidden_states.shape)


# speedup vs baseline: 3.6452x; 1.0908x over previous
import jax
import jax.numpy as jnp
from jax.experimental import pallas as pl
from jax.experimental.pallas import tpu as pltpu

B, S, H = 1, 2048, 1024
E = 8
I = 512
ISH = 1024
T = B * S
BT = 1024
NTB = T // BT


def _dot_nt(a, b):
    return jax.lax.dot_general(a, b, (((1,), (1,)), ((), ())),
                               preferred_element_type=jnp.float32)


def _moe_kernel(x_ref, gatew_ref, wg_ref, wu_ref, wd_ref,
                shg_ref, shu_ref, shd_ref, out_ref, comb_ref):
    e = pl.program_id(0)

    @pl.when(e == 0)
    def _gate_and_shared():
        for tb in range(NTB):
            sl = pl.ds(tb * BT, BT)
            xb = x_ref[sl, :]
            logits = _dot_nt(xb, gatew_ref[...])
            m = jnp.max(logits, axis=-1, keepdims=True)
            ex = jnp.exp(logits - m)
            scores = ex / jnp.sum(ex, axis=-1, keepdims=True)
            eidx = jax.lax.broadcasted_iota(jnp.int32, (BT, E), 1)
            i1 = jnp.argmax(scores, axis=-1)
            m1 = jnp.max(scores, axis=-1)
            masked = jnp.where(eidx == i1[:, None], -jnp.inf, scores)
            i2 = jnp.argmax(masked, axis=-1)
            m2 = jnp.max(masked, axis=-1)
            denom = m1 + m2 + 1e-20
            w1 = (m1 / denom)[:, None]
            w2 = (m2 / denom)[:, None]
            comb = (jnp.where(eidx == i1[:, None], w1, 0.0)
                    + jnp.where(eidx == i2[:, None], w2, 0.0))
            comb_ref[sl, :] = comb
            xb16 = xb.astype(jnp.bfloat16)
            sg = _dot_nt(xb16, shg_ref[...].astype(jnp.bfloat16))
            su = _dot_nt(xb16, shu_ref[...].astype(jnp.bfloat16))
            act = (sg * jax.lax.logistic(sg)) * su
            ys = _dot_nt(act.astype(jnp.bfloat16),
                         shd_ref[...].astype(jnp.bfloat16))
            out_ref[sl, :] = ys

    @pl.when(e > 0)
    def _routed():
        wg = wg_ref[0].astype(jnp.bfloat16)
        wu = wu_ref[0].astype(jnp.bfloat16)
        wd = wd_ref[0].astype(jnp.bfloat16)
        onehot = (jax.lax.broadcasted_iota(jnp.int32, (E, 1), 0)
                  == e - 1).astype(jnp.float32)
        for tb in range(NTB):
            sl = pl.ds(tb * BT, BT)
            xb16 = x_ref[sl, :].astype(jnp.bfloat16)
            g = _dot_nt(xb16, wg)
            u = _dot_nt(xb16, wu)
            a = (g * jax.lax.logistic(g)) * u
            eo = _dot_nt(a.astype(jnp.bfloat16), wd)
            cw = jnp.dot(comb_ref[sl, :], onehot,
                         preferred_element_type=jnp.float32)
            out_ref[sl, :] += cw * eo


def kernel(hidden_states, gate_w, w_gate, w_up, w_down, sh_gate, sh_up, sh_down):
    x = hidden_states.reshape(T, H)
    out = pl.pallas_call(
        _moe_kernel,
        grid=(E + 1,),
        in_specs=[
            pl.BlockSpec((T, H), lambda e: (0, 0)),
            pl.BlockSpec((E, H), lambda e: (0, 0)),
            pl.BlockSpec((1, I, H), lambda e: (jnp.maximum(e - 1, 0), 0, 0)),
            pl.BlockSpec((1, I, H), lambda e: (jnp.maximum(e - 1, 0), 0, 0)),
            pl.BlockSpec((1, H, I), lambda e: (jnp.maximum(e - 1, 0), 0, 0)),
            pl.BlockSpec((ISH, H), lambda e: (0, 0)),
            pl.BlockSpec((ISH, H), lambda e: (0, 0)),
            pl.BlockSpec((H, ISH), lambda e: (0, 0)),
        ],
        out_specs=pl.BlockSpec((T, H), lambda e: (0, 0)),
        out_shape=jax.ShapeDtypeStruct((T, H), jnp.float32),
        scratch_shapes=[pltpu.VMEM((T, E), jnp.float32)],
    )(x, gate_w, w_gate, w_up, w_down, sh_gate, sh_up, sh_down)
    return out.reshape(hidden_states.shape)
